# baseline (speedup 1.0000x reference)
import jax
import jax.numpy as jnp
from jax import lax
from jax.experimental import pallas as pl
from jax.experimental.pallas import tpu as pltpu
from jax.experimental.pallas import tpu_sc as plsc

_NC = 2
_NS = 16
_L = 16
_B = 128
_R = 64


def _cheb_step(is_first, NP, CAP, H):
    NH = NP // _NS
    NFC = NH // _R
    NCH = CAP // _B
    CAPT = _NS * CAP
    assert NH % _R == 0 and CAP % _B == 0

    mesh = plsc.VectorSubcoreMesh(
        core_axis_name="c", subcore_axis_name="s",
        num_cores=_NC, num_subcores=_NS)

    def body(tp1, tp2, oacc_in, cols2, rowsP, valsP, thA, thB,
             t_out, oacc_out,
             idx_v, row_v, val_v, gbuf, pbuf, obuf,
             thA_v, thB_v, acc_v, sem):
        c = lax.axis_index("c")
        s = lax.axis_index("s")

        def zrow(r, carry):
            for j in range(H // _L):
                acc_v[r, pl.ds(j * _L, _L)] = jnp.zeros((_L,), jnp.float32)
            return carry
        lax.fori_loop(0, NH, zrow, 0)

        def chunk(i, carry):
            off = s * CAP + i * _B
            pltpu.sync_copy(cols2.at[pl.ds(c * CAPT + off, _B)], idx_v)
            pltpu.sync_copy(rowsP.at[pl.ds(off, _B)], row_v)
            pltpu.sync_copy(valsP.at[pl.ds(off, _B)], val_v)
            pltpu.async_copy(tp1.at[idx_v], gbuf, sem).wait()

            def eadd(g, ecarry):
                vvec = val_v[pl.ds(g * _L, _L)]
                rvec = row_v[pl.ds(g * _L, _L)]
                for i2 in range(_L):
                    v = vvec[i2]
                    rr = rvec[i2]
                    e = g * _L + i2
                    for j in range(H // _L):
                        sl = pl.ds(j * _L, _L)
                        plsc.addupdate(acc_v.at[rr, sl], gbuf[e, sl] * v)
                return ecarry
            lax.fori_loop(0, _B // _L, eadd, 0)
            return carry
        lax.fori_loop(0, NCH, chunk, 0)

        if is_first:
            pltpu.sync_copy(thA.at[c], thA_v)
        pltpu.sync_copy(thB.at[c], thB_v)
        for ci in range(NFC):
            rloc = ci * _R
            g = c * NP + s * NH + rloc
            pltpu.sync_copy(tp2.at[pl.ds(g, _R)], pbuf)
            if not is_first:
                pltpu.sync_copy(oacc_in.at[pl.ds(g, _R)], obuf)

            def frow(r, carry):
                for j in range(H // _L):
                    sl = pl.ds(j * _L, _L)
                    a = acc_v[rloc + r, sl]
                    if is_first:
                        t = a
                        o = thA_v[sl] * pbuf[r, sl] + thB_v[sl] * t
                    else:
                        t = 2.0 * a - pbuf[r, sl]
                        o = obuf[r, sl] + thB_v[sl] * t
                    gbuf[r, sl] = t
                    obuf[r, sl] = o
                return carry
            lax.fori_loop(0, _R, frow, 0)
            pltpu.sync_copy(gbuf.at[pl.ds(0, _R)], t_out.at[pl.ds(g, _R)])
            pltpu.sync_copy(obuf, oacc_out.at[pl.ds(g, _R)])

    out_t = [jax.ShapeDtypeStruct((_NC * NP, H), jnp.float32),
             jax.ShapeDtypeStruct((_NC * NP, H), jnp.float32)]
    scratch = [
        pltpu.VMEM((_B,), jnp.int32),
        pltpu.VMEM((_B,), jnp.int32),
        pltpu.VMEM((_B,), jnp.float32),
        pltpu.VMEM((_B, H), jnp.float32),
        pltpu.VMEM((_R, H), jnp.float32),
        pltpu.VMEM((_R, H), jnp.float32),
        pltpu.VMEM((H,), jnp.float32),
        pltpu.VMEM((H,), jnp.float32),
        pltpu.VMEM((NP // _NS, H), jnp.float32),
        pltpu.SemaphoreType.DMA,
    ]
    return pl.kernel(body, out_type=out_t, mesh=mesh, scratch_types=scratch,
                     name="cheb_first" if is_first else "cheb_step")


@jax.jit
def kernel(x, slap_vals, theta, slap_rows, slap_cols):
    N, D = x.shape
    K = theta.shape[0]
    E = slap_rows.shape[0]
    H = D // _NC

    NP = -(-N // (_NS * _R)) * (_NS * _R)
    NH = NP // _NS
    mean = E * NH / N
    CAP = int(-(-(mean + 9.5 * (mean ** 0.5)) // _B) * _B)

    bucket = slap_rows // NH
    order = jnp.argsort(bucket)
    srows = slap_rows[order]
    scols = slap_cols[order]
    svals = slap_vals[order]
    sbucket = bucket[order]
    starts = jnp.searchsorted(sbucket, jnp.arange(_NS, dtype=jnp.int32))
    counts = jnp.append(starts[1:], E) - starts

    slot = jnp.arange(_NS * CAP, dtype=jnp.int32)
    b = slot // CAP
    r = slot - b * CAP
    src = starts[b] + r
    valid = r < counts[b]
    srcc = jnp.where(valid, jnp.minimum(src, E - 1), 0)
    rowsP = jnp.where(valid, jnp.take(srows, srcc) - b * NH, 0)
    colsP = jnp.where(valid, jnp.take(scols, srcc), 0)
    valsP = jnp.where(valid, jnp.take(svals, srcc), 0.0)
    cols2 = jnp.concatenate([colsP, colsP + NP])
    th = theta.reshape(K, _NC, H)

    x_st = x.reshape(N, _NC, H).transpose(1, 0, 2)
    x_st = jnp.pad(x_st, ((0, 0), (0, NP - N), (0, 0))).reshape(_NC * NP, H)

    first = _cheb_step(True, NP, CAP, H)
    step = _cheb_step(False, NP, CAP, H)

    t1, oacc = first(x_st, x_st, x_st, cols2, rowsP, valsP, th[0], th[1])
    tm2, tm1 = x_st, t1
    for k in range(2, K):
        tk, oacc = step(tm1, tm2, oacc, cols2, rowsP, valsP, th[k], th[k])
        tm2, tm1 = tm1, tk

    out = oacc.reshape(_NC, NP, H)[:, :N]
    return out.transpose(1, 0, 2).reshape(N, D)


# speedup vs baseline: 1.2517x; 1.2517x over previous
import jax
import jax.numpy as jnp
from jax import lax
from jax.experimental import pallas as pl
from jax.experimental.pallas import tpu as pltpu
from jax.experimental.pallas import tpu_sc as plsc

_NC = 2
_NS = 16
_L = 16
_B = 128
_R = 32


def _cheb_step(is_first, NP, CAP, H):
    NH = NP // _NS
    NFC = NH // _R
    NCH = CAP // _B
    EDC = 2 * _B
    CL = _NS * NCH * EDC + EDC
    assert NH % _R == 0 and CAP % _B == 0 and NCH % 2 == 0

    mesh = plsc.VectorSubcoreMesh(
        core_axis_name="c", subcore_axis_name="s",
        num_cores=_NC, num_subcores=_NS)

    def body(tp1, tp2, oacc_in, edata, valsH, thA, thB,
             t_out, oacc_out,
             ebuf0, ebuf1, vbuf0, vbuf1, gbuf0, gbuf1, pbuf, obuf,
             thA_v, thB_v, acc_v, sem0, sem1):
        c = lax.axis_index("c")
        s = lax.axis_index("s")
        ebufs = (ebuf0, ebuf1)
        vbufs = (vbuf0, vbuf1)
        gbufs = (gbuf0, gbuf1)
        sems = (sem0, sem1)

        def zrow(r, carry):
            for j in range(H // _L):
                acc_v[r, pl.ds(j * _L, _L)] = jnp.zeros((_L,), jnp.float32)
            return carry
        lax.fori_loop(0, NH, zrow, 0)

        def load_e(i, b):
            base = c * CL + (s * NCH + i) * EDC
            pltpu.sync_copy(edata.at[pl.ds(base, EDC)], ebufs[b])
            vbase = (s * NCH + i) * _B
            pltpu.sync_copy(valsH.at[pl.ds(vbase, _B)], vbufs[b])

        def start_g(i, b):
            pltpu.async_copy(tp1.at[ebufs[b].at[pl.ds(0, _B)]],
                             gbufs[b], sems[b])

        def wait_g(b):
            pltpu.make_async_copy(tp1.at[pl.ds(0, _B)],
                                  gbufs[b], sems[b]).wait()

        def compute(b):
            ebuf = ebufs[b]
            gbuf = gbufs[b]

            vbuf = vbufs[b]

            def eadd(g, ecarry):
                rvec = ebuf[pl.ds(_B + g * _L, _L)]
                vvec = vbuf[pl.ds(g * _L, _L)]
                for i2 in range(_L):
                    v = vvec[i2]
                    rr = rvec[i2]
                    e = g * _L + i2
                    for j in range(H // _L):
                        sl = pl.ds(j * _L, _L)
                        plsc.addupdate(acc_v.at[rr, sl], gbuf[e, sl] * v)
                return ecarry
            lax.fori_loop(0, _B // _L, eadd, 0)

        load_e(0, 0)
        start_g(0, 0)

        def outer(io, carry):
            for b in range(2):
                cur = io * 2 + b
                nb = 1 - b
                load_e(cur + 1, nb)
                start_g(cur + 1, nb)
                wait_g(b)
                compute(b)
            return carry
        lax.fori_loop(0, NCH // 2, outer, 0)
        wait_g(0)

        if is_first:
            pltpu.sync_copy(thA.at[c], thA_v)
        pltpu.sync_copy(thB.at[c], thB_v)
        for ci in range(NFC):
            rloc = ci * _R
            g = c * NP + s * NH + rloc
            pltpu.sync_copy(tp2.at[pl.ds(g, _R)], pbuf)
            if not is_first:
                pltpu.sync_copy(oacc_in.at[pl.ds(g, _R)], obuf)

            def frow(r, carry):
                for j in range(H // _L):
                    sl = pl.ds(j * _L, _L)
                    a = acc_v[rloc + r, sl]
                    if is_first:
                        t = a
                        o = thA_v[sl] * pbuf[r, sl] + thB_v[sl] * t
                    else:
                        t = 2.0 * a - pbuf[r, sl]
                        o = obuf[r, sl] + thB_v[sl] * t
                    gbuf0[r, sl] = t
                    obuf[r, sl] = o
                return carry
            lax.fori_loop(0, _R, frow, 0)
            pltpu.sync_copy(gbuf0.at[pl.ds(0, _R)], t_out.at[pl.ds(g, _R)])
            pltpu.sync_copy(obuf, oacc_out.at[pl.ds(g, _R)])

    out_t = [jax.ShapeDtypeStruct((_NC * NP, H), jnp.float32),
             jax.ShapeDtypeStruct((_NC * NP, H), jnp.float32)]
    scratch = [
        pltpu.VMEM((EDC,), jnp.int32),
        pltpu.VMEM((EDC,), jnp.int32),
        pltpu.VMEM((_B,), jnp.float32),
        pltpu.VMEM((_B,), jnp.float32),
        pltpu.VMEM((_B, H), jnp.float32),
        pltpu.VMEM((_B, H), jnp.float32),
        pltpu.VMEM((_R, H), jnp.float32),
        pltpu.VMEM((_R, H), jnp.float32),
        pltpu.VMEM((H,), jnp.float32),
        pltpu.VMEM((H,), jnp.float32),
        pltpu.VMEM((NP // _NS, H), jnp.float32),
        pltpu.SemaphoreType.DMA,
        pltpu.SemaphoreType.DMA,
    ]
    return pl.kernel(body, out_type=out_t, mesh=mesh, scratch_types=scratch,
                     name="cheb_first" if is_first else "cheb_step")


@jax.jit
def kernel(x, slap_vals, theta, slap_rows, slap_cols):
    N, D = x.shape
    K = theta.shape[0]
    E = slap_rows.shape[0]
    H = D // _NC

    NP = -(-N // (_NS * _R)) * (_NS * _R)
    NH = NP // _NS
    mean = E * NH / N
    CAP = int(-(-(mean + 9.5 * (mean ** 0.5)) // (2 * _B)) * (2 * _B))

    bucket = slap_rows // NH
    order = jnp.argsort(bucket)
    srows = slap_rows[order]
    scols = slap_cols[order]
    svals = slap_vals[order]
    sbucket = bucket[order]
    starts = jnp.searchsorted(sbucket, jnp.arange(_NS, dtype=jnp.int32))
    counts = jnp.append(starts[1:], E) - starts

    slot = jnp.arange(_NS * CAP, dtype=jnp.int32)
    b = slot // CAP
    r = slot - b * CAP
    src = starts[b] + r
    valid = r < counts[b]
    srcc = jnp.where(valid, jnp.minimum(src, E - 1), 0)
    rowsP = jnp.where(valid, jnp.take(srows, srcc) - b * NH, 0)
    colsP = jnp.where(valid, jnp.take(scols, srcc), 0)
    valsP = jnp.where(valid, jnp.take(svals, srcc), 0.0)
    valsH = jnp.concatenate([valsP, jnp.zeros((_B,), jnp.float32)])

    M = _NS * CAP // _B
    slack = jnp.zeros((1, 2, _B), jnp.int32)
    eparts = []
    for ct in range(_NC):
        ed = jnp.stack([(colsP + ct * NP).reshape(M, _B),
                        rowsP.reshape(M, _B)], axis=1)
        eparts.append(jnp.concatenate([ed, slack]).reshape(-1))
    edata = jnp.concatenate(eparts)
    th = theta.reshape(K, _NC, H)

    x_st = x.reshape(N, _NC, H).transpose(1, 0, 2)
    x_st = jnp.pad(x_st, ((0, 0), (0, NP - N), (0, 0))).reshape(_NC * NP, H)

    first = _cheb_step(True, NP, CAP, H)
    step = _cheb_step(False, NP, CAP, H)

    t1, oacc = first(x_st, x_st, x_st, edata, valsH, th[0], th[1])
    tm2, tm1 = x_st, t1
    for k in range(2, K):
        tk, oacc = step(tm1, tm2, oacc, edata, valsH, th[k], th[k])
        tm2, tm1 = tm1, tk

    out = oacc.reshape(_NC, NP, H)[:, :N]
    return out.transpose(1, 0, 2).reshape(N, D)
